# initial kernel scaffold (unmeasured)
import jax
import jax.numpy as jnp
from jax import lax
from jax.experimental import pallas as pl
from jax.experimental.pallas import tpu as pltpu

N_DEV = 32


def kernel(x, w_mat, scale_x, scale_w):
    k_total, k_per = x.shape
    _, n = w_mat.shape
    m_per = k_total // N_DEV

    def body(x_ref, w_ref, sx_ref, sw_ref, out_ref,
             xrow_ref, send_sems, recv_sems):
        my = lax.axis_index("i")

        xrow_ref[:, pl.ds(my * k_per, k_per)] = x_ref[pl.ds(my * m_per, m_per), :]

        rdmas = []
        for o in range(1, N_DEV):
            d = (my + o) % N_DEV
            rdma = pltpu.make_async_remote_copy(
                src_ref=x_ref.at[pl.ds(d * m_per, m_per), :],
                dst_ref=xrow_ref.at[:, pl.ds(my * k_per, k_per)],
                send_sem=send_sems.at[o],
                recv_sem=recv_sems.at[o],
                device_id=(d,),
                device_id_type=pl.DeviceIdType.MESH,
            )
            rdma.start()
            rdmas.append(rdma)

        for o in range(1, N_DEV):
            s = (my - o) % N_DEV
            recv = pltpu.make_async_remote_copy(
                src_ref=x_ref.at[pl.ds(0, m_per), :],
                dst_ref=xrow_ref.at[:, pl.ds(s * k_per, k_per)],
                send_sem=send_sems.at[o],
                recv_sem=recv_sems.at[o],
                device_id=(s,),
                device_id_type=pl.DeviceIdType.MESH,
            )
            recv.wait_recv()

        acc = jnp.dot(xrow_ref[:, :], w_ref[:, :],
                      preferred_element_type=jnp.float32)
        y = acc * (sx_ref[0] * sw_ref[0])
        out_ref[:, :] = y * jax.nn.sigmoid(y)

        for rdma in rdmas:
            rdma.wait_send()

    return pl.pallas_call(
        body,
        out_shape=jax.ShapeDtypeStruct((m_per, n), jnp.float32),
        in_specs=[
            pl.BlockSpec(memory_space=pltpu.VMEM),
            pl.BlockSpec(memory_space=pltpu.VMEM),
            pl.BlockSpec(memory_space=pltpu.SMEM),
            pl.BlockSpec(memory_space=pltpu.SMEM),
        ],
        out_specs=pl.BlockSpec(memory_space=pltpu.VMEM),
        scratch_shapes=[
            pltpu.VMEM((m_per, k_total), x.dtype),
            pltpu.SemaphoreType.DMA((N_DEV,)),
            pltpu.SemaphoreType.DMA((N_DEV,)),
        ],
        compiler_params=pltpu.CompilerParams(collective_id=0),
    )(x, w_mat, scale_x, scale_w)


# baseline (device time: 50128 ns/iter reference)
import jax
import jax.numpy as jnp
from jax import lax
from jax.experimental import pallas as pl
from jax.experimental.pallas import tpu as pltpu

N_DEV = 32
N_BLK = 512


def kernel(x, w_mat, scale_x, scale_w):
    k_total, k_per = x.shape
    _, n = w_mat.shape
    m_per = k_total // N_DEV
    n_steps = n // N_BLK

    def body(x_ref, w_ref, sx_ref, sw_ref, out_ref,
             xrow_ref, send_sems, recv_sems):
        j = pl.program_id(0)
        my = lax.axis_index("i")

        @pl.when(j == 0)
        def _a2a():
            xrow_ref[:, pl.ds(my * k_per, k_per)] = (
                x_ref[pl.ds(my * m_per, m_per), :])

            rdmas = []
            for o in range(1, N_DEV):
                d = (my + o) % N_DEV
                rdma = pltpu.make_async_remote_copy(
                    src_ref=x_ref.at[pl.ds(d * m_per, m_per), :],
                    dst_ref=xrow_ref.at[:, pl.ds(my * k_per, k_per)],
                    send_sem=send_sems.at[o],
                    recv_sem=recv_sems.at[o],
                    device_id=(d,),
                    device_id_type=pl.DeviceIdType.MESH,
                )
                rdma.start()
                rdmas.append(rdma)

            for o in range(1, N_DEV):
                s = (my - o) % N_DEV
                recv = pltpu.make_async_remote_copy(
                    src_ref=x_ref.at[pl.ds(0, m_per), :],
                    dst_ref=xrow_ref.at[:, pl.ds(s * k_per, k_per)],
                    send_sem=send_sems.at[o],
                    recv_sem=recv_sems.at[o],
                    device_id=(s,),
                    device_id_type=pl.DeviceIdType.MESH,
                )
                recv.wait_recv()

            for rdma in rdmas:
                rdma.wait_send()

        acc = jnp.dot(xrow_ref[:, :], w_ref[:, :],
                      preferred_element_type=jnp.float32)
        y = acc * (sx_ref[0] * sw_ref[0])
        out_ref[:, :] = y * jax.nn.sigmoid(y)

    return pl.pallas_call(
        body,
        grid=(n_steps,),
        out_shape=jax.ShapeDtypeStruct((m_per, n), jnp.float32),
        in_specs=[
            pl.BlockSpec((k_total, k_per), lambda j: (0, 0)),
            pl.BlockSpec((k_total, N_BLK), lambda j: (0, j)),
            pl.BlockSpec(memory_space=pltpu.SMEM),
            pl.BlockSpec(memory_space=pltpu.SMEM),
        ],
        out_specs=pl.BlockSpec((m_per, N_BLK), lambda j: (0, j)),
        scratch_shapes=[
            pltpu.VMEM((m_per, k_total), x.dtype),
            pltpu.SemaphoreType.DMA((N_DEV,)),
            pltpu.SemaphoreType.DMA((N_DEV,)),
        ],
    )(x, w_mat, scale_x, scale_w)


# device time: 33094 ns/iter; 1.5147x vs baseline; 1.5147x over previous
import jax
import jax.numpy as jnp
from jax import lax
from jax.experimental import pallas as pl
from jax.experimental.pallas import tpu as pltpu

N_DEV = 32
N_BLK = 512
FP8 = jnp.float8_e5m2


def kernel(x, w_mat, scale_x, scale_w):
    k_total, k_per = x.shape
    _, n = w_mat.shape
    m_per = k_total // N_DEV
    n_steps = n // N_BLK

    def body(x_ref, w_ref, sx_ref, sw_ref, out_ref,
             xq_ref, xrow_ref, send_sems, recv_sems):
        j = pl.program_id(0)
        my = lax.axis_index("i")

        @pl.when(j == 0)
        def _a2a():
            xq_ref[:, :] = x_ref[:, :].astype(FP8)

            xrow_ref[:, pl.ds(my * k_per, k_per)] = (
                xq_ref[pl.ds(my * m_per, m_per), :])

            rdmas = []
            for o in range(1, N_DEV):
                d = (my + o) % N_DEV
                rdma = pltpu.make_async_remote_copy(
                    src_ref=xq_ref.at[pl.ds(d * m_per, m_per), :],
                    dst_ref=xrow_ref.at[:, pl.ds(my * k_per, k_per)],
                    send_sem=send_sems.at[o],
                    recv_sem=recv_sems.at[o],
                    device_id=(d,),
                    device_id_type=pl.DeviceIdType.MESH,
                )
                rdma.start()
                rdmas.append(rdma)

            for o in range(1, N_DEV):
                s = (my - o) % N_DEV
                recv = pltpu.make_async_remote_copy(
                    src_ref=xq_ref.at[pl.ds(0, m_per), :],
                    dst_ref=xrow_ref.at[:, pl.ds(s * k_per, k_per)],
                    send_sem=send_sems.at[o],
                    recv_sem=recv_sems.at[o],
                    device_id=(s,),
                    device_id_type=pl.DeviceIdType.MESH,
                )
                recv.wait_recv()

            for rdma in rdmas:
                rdma.wait_send()

        wq = w_ref[:, :].astype(FP8)
        acc = jnp.dot(xrow_ref[:, :], wq, preferred_element_type=jnp.float32)
        y = acc * (sx_ref[0] * sw_ref[0])
        out_ref[:, :] = y * jax.nn.sigmoid(y)

    return pl.pallas_call(
        body,
        grid=(n_steps,),
        out_shape=jax.ShapeDtypeStruct((m_per, n), jnp.float32),
        in_specs=[
            pl.BlockSpec((k_total, k_per), lambda j: (0, 0)),
            pl.BlockSpec((k_total, N_BLK), lambda j: (0, j)),
            pl.BlockSpec(memory_space=pltpu.SMEM),
            pl.BlockSpec(memory_space=pltpu.SMEM),
        ],
        out_specs=pl.BlockSpec((m_per, N_BLK), lambda j: (0, j)),
        scratch_shapes=[
            pltpu.VMEM((k_total, k_per), FP8),
            pltpu.VMEM((m_per, k_total), FP8),
            pltpu.SemaphoreType.DMA((N_DEV,)),
            pltpu.SemaphoreType.DMA((N_DEV,)),
        ],
    )(x, w_mat, scale_x, scale_w)


# device time: 29687 ns/iter; 1.6886x vs baseline; 1.1148x over previous
import jax
import jax.numpy as jnp
from jax import lax
from jax.experimental import pallas as pl
from jax.experimental.pallas import tpu as pltpu

N_DEV = 32
K_BLK = 512
FP8 = jnp.float8_e5m2


def kernel(x, w_mat, scale_x, scale_w):
    k_total, k_per = x.shape
    _, n = w_mat.shape
    m_per = k_total // N_DEV
    k_steps = k_total // K_BLK
    src_per_chunk = K_BLK // k_per

    def body(x_ref, w_ref, sx_ref, sw_ref, out_ref,
             xq_ref, xrow_ref, send_sems, recv_sems):
        j = pl.program_id(0)
        my = lax.axis_index("i")

        @pl.when(j == 0)
        def _start():
            barrier_sem = pltpu.get_barrier_semaphore()
            for o in range(1, N_DEV):
                pl.semaphore_signal(
                    barrier_sem, inc=1,
                    device_id=((my + o) % N_DEV,),
                    device_id_type=pl.DeviceIdType.MESH,
                )
            xq_ref[:, :] = x_ref[:, :].astype(FP8)
            xrow_ref[:, pl.ds(my * k_per, k_per)] = (
                xq_ref[pl.ds(my * m_per, m_per), :])
            pl.semaphore_wait(barrier_sem, N_DEV - 1)

            for o in range(1, N_DEV):
                d = (my + o) % N_DEV
                pltpu.make_async_remote_copy(
                    src_ref=xq_ref.at[pl.ds(d * m_per, m_per), :],
                    dst_ref=xrow_ref.at[:, pl.ds(my * k_per, k_per)],
                    send_sem=send_sems.at[o],
                    recv_sem=recv_sems.at[o],
                    device_id=(d,),
                    device_id_type=pl.DeviceIdType.MESH,
                ).start()

        for s in range(N_DEV):
            ck = s // src_per_chunk
            o = (my - s) % N_DEV

            @pl.when((j == ck) & (s != my))
            def _wait(s=s, o=o):
                pltpu.make_async_remote_copy(
                    src_ref=xq_ref.at[pl.ds(0, m_per), :],
                    dst_ref=xrow_ref.at[:, pl.ds(s * k_per, k_per)],
                    send_sem=send_sems.at[o],
                    recv_sem=recv_sems.at[o],
                    device_id=(s,),
                    device_id_type=pl.DeviceIdType.MESH,
                ).wait_recv()

        wq = w_ref[:, :].astype(FP8)
        partial = jnp.dot(xrow_ref[:, pl.ds(j * K_BLK, K_BLK)], wq,
                          preferred_element_type=jnp.float32)

        @pl.when(j == 0)
        def _init():
            out_ref[:, :] = partial

        @pl.when(j > 0)
        def _acc():
            out_ref[:, :] = out_ref[:, :] + partial

        @pl.when(j == k_steps - 1)
        def _finish():
            for o in range(1, N_DEV):
                pltpu.make_async_remote_copy(
                    src_ref=xq_ref.at[pl.ds(0, m_per), :],
                    dst_ref=xrow_ref.at[:, pl.ds(0, k_per)],
                    send_sem=send_sems.at[o],
                    recv_sem=recv_sems.at[o],
                    device_id=(my,),
                    device_id_type=pl.DeviceIdType.MESH,
                ).wait_send()
            y = out_ref[:, :] * (sx_ref[0] * sw_ref[0])
            out_ref[:, :] = y * jax.nn.sigmoid(y)

    return pl.pallas_call(
        body,
        grid=(k_steps,),
        out_shape=jax.ShapeDtypeStruct((m_per, n), jnp.float32),
        in_specs=[
            pl.BlockSpec((k_total, k_per), lambda j: (0, 0)),
            pl.BlockSpec((K_BLK, n), lambda j: (j, 0)),
            pl.BlockSpec(memory_space=pltpu.SMEM),
            pl.BlockSpec(memory_space=pltpu.SMEM),
        ],
        out_specs=pl.BlockSpec((m_per, n), lambda j: (0, 0)),
        scratch_shapes=[
            pltpu.VMEM((k_total, k_per), FP8),
            pltpu.VMEM((m_per, k_total), FP8),
            pltpu.SemaphoreType.DMA((N_DEV,)),
            pltpu.SemaphoreType.DMA((N_DEV,)),
        ],
        compiler_params=pltpu.CompilerParams(collective_id=0),
    )(x, w_mat, scale_x, scale_w)


# device time: 19027 ns/iter; 2.6346x vs baseline; 1.5603x over previous
import jax
import jax.numpy as jnp
from jax import lax
from jax.experimental import pallas as pl
from jax.experimental.pallas import tpu as pltpu

N_DEV = 32
K_BLK = 512
NBUF = 4
FP8 = jnp.float8_e5m2

DO_A2A = True
DO_ACKS = True
ROTATE = False
LOOKAHEAD = 8
SELF_BARRIER = True
ACC_VALUE = True


def kernel(x, w_mat, scale_x, scale_w):
    k_total, k_per = x.shape
    _, n = w_mat.shape
    m_per = k_total // N_DEV
    k_steps = k_total // K_BLK
    spc = K_BLK // k_per

    def body(x_ref, w_hbm, sx_ref, sw_ref, out_ref,
             xq_ref, xrow_ref, wbuf_ref, wsems,
             send_sems, recv_sems, ack_sems):
        my = lax.axis_index("i")
        my_chunk = my // spc if ROTATE else my * 0

        barrier_sem = pltpu.get_barrier_semaphore()
        if SELF_BARRIER:
            pl.semaphore_signal(barrier_sem, inc=1)
        else:
            for nbr in ((my + 1) % N_DEV, (my - 1) % N_DEV):
                pl.semaphore_signal(
                    barrier_sem, inc=1, device_id=(nbr,),
                    device_id_type=pl.DeviceIdType.MESH,
                )

        if DO_ACKS:
            for o in range(1, N_DEV):
                pl.semaphore_signal(
                    ack_sems.at[N_DEV - o], inc=1,
                    device_id=((my + o) % N_DEV,),
                    device_id_type=pl.DeviceIdType.MESH,
                )

        def fetch(i):
            c = (my_chunk + i) % k_steps
            pltpu.make_async_copy(
                w_hbm.at[pl.ds(c * K_BLK, K_BLK), :],
                wbuf_ref.at[i % NBUF],
                wsems.at[i % NBUF],
            ).start()

        for i in range(NBUF):
            fetch(i)

        xq_ref[:, :] = x_ref[:, :].astype(FP8)
        xrow_ref[:, pl.ds(my * k_per, k_per)] = (
            xq_ref[pl.ds(my * m_per, m_per), :])

        pl.semaphore_wait(barrier_sem, 1 if SELF_BARRIER else 2)

        def send_group(g):
            for t in range(spc):
                d = ((my_chunk - g) % k_steps) * spc + t
                o = (d - my) % N_DEV

                @pl.when(d != my)
                def _send(d=d, o=o):
                    if DO_ACKS:
                        pl.semaphore_wait(ack_sems.at[o], 1)
                    pltpu.make_async_remote_copy(
                        src_ref=xq_ref.at[pl.ds(d * m_per, m_per), :],
                        dst_ref=xrow_ref.at[:, pl.ds(my * k_per, k_per)],
                        send_sem=send_sems.at[o],
                        recv_sem=recv_sems.at[o],
                        device_id=(d,),
                        device_id_type=pl.DeviceIdType.MESH,
                    ).start()

        if DO_A2A:
            for g in range(LOOKAHEAD):
                send_group(g)
        else:
            for s in range(N_DEV):
                xrow_ref[:, pl.ds(s * k_per, k_per)] = (
                    xq_ref[pl.ds(s * m_per, m_per), :])

        acc = None
        for i in range(k_steps):
            c = (my_chunk + i) % k_steps

            if DO_A2A:
                for t in range(spc):
                    s = c * spc + t
                    o = (my - s) % N_DEV

                    @pl.when(s != my)
                    def _wait(s=s, o=o):
                        pltpu.make_async_remote_copy(
                            src_ref=xq_ref.at[pl.ds(0, m_per), :],
                            dst_ref=xrow_ref.at[:, pl.ds(s * k_per, k_per)],
                            send_sem=send_sems.at[o],
                            recv_sem=recv_sems.at[o],
                            device_id=(s,),
                            device_id_type=pl.DeviceIdType.MESH,
                        ).wait_recv()

            pltpu.make_async_copy(
                w_hbm.at[pl.ds(c * K_BLK, K_BLK), :],
                wbuf_ref.at[i % NBUF],
                wsems.at[i % NBUF],
            ).wait()
            wq = wbuf_ref[i % NBUF].astype(FP8)
            partial = jnp.dot(xrow_ref[:, pl.ds(c * K_BLK, K_BLK)], wq,
                              preferred_element_type=jnp.float32)
            if ACC_VALUE:
                acc = partial if i == 0 else acc + partial
            elif i == 0:
                out_ref[:, :] = partial
            else:
                out_ref[:, :] = out_ref[:, :] + partial

            if i + NBUF < k_steps:
                fetch(i + NBUF)
            if DO_A2A and i + LOOKAHEAD < k_steps:
                send_group(i + LOOKAHEAD)

        if DO_A2A:
            for o in range(1, N_DEV):
                pltpu.make_async_remote_copy(
                    src_ref=xq_ref.at[pl.ds(0, m_per), :],
                    dst_ref=xrow_ref.at[:, pl.ds(0, k_per)],
                    send_sem=send_sems.at[o],
                    recv_sem=recv_sems.at[o],
                    device_id=(my,),
                    device_id_type=pl.DeviceIdType.MESH,
                ).wait_send()

        y = (acc if ACC_VALUE else out_ref[:, :]) * (sx_ref[0] * sw_ref[0])
        out_ref[:, :] = y * jax.nn.sigmoid(y)

    return pl.pallas_call(
        body,
        out_shape=jax.ShapeDtypeStruct((m_per, n), jnp.float32),
        in_specs=[
            pl.BlockSpec(memory_space=pltpu.VMEM),
            pl.BlockSpec(memory_space=pl.ANY),
            pl.BlockSpec(memory_space=pltpu.SMEM),
            pl.BlockSpec(memory_space=pltpu.SMEM),
        ],
        out_specs=pl.BlockSpec(memory_space=pltpu.VMEM),
        scratch_shapes=[
            pltpu.VMEM((k_total, k_per), FP8),
            pltpu.VMEM((m_per, k_total), FP8),
            pltpu.VMEM((NBUF, K_BLK, n), jnp.float32),
            pltpu.SemaphoreType.DMA((NBUF,)),
            pltpu.SemaphoreType.DMA((N_DEV,)),
            pltpu.SemaphoreType.DMA((N_DEV,)),
            pltpu.SemaphoreType.REGULAR((N_DEV,)),
        ],
        compiler_params=pltpu.CompilerParams(collective_id=0),
    )(x, w_mat, scale_x, scale_w)
